# baseline (speedup 1.0000x reference)
import numpy as np

import jax
import jax.numpy as jnp
from jax.experimental import pallas as pl

_N_TERMS = 64
_N_OUT = 4
_MASKS = [np.ones(_N_TERMS, dtype=bool) for _ in range(_N_OUT)]

_ROW_BLOCK = 20000


def _gather_cols_kernel(x_ref, o_ref):
    o_ref[...] = x_ref[...]


def _masked_gather(thetas, cols):
    n, _ = thetas.shape
    w = int(cols.shape[0])
    grid = n // _ROW_BLOCK
    return pl.pallas_call(
        _gather_cols_kernel,
        grid=(grid,),
        in_specs=[pl.BlockSpec((_ROW_BLOCK, w), lambda i: (i, 0))],
        out_specs=pl.BlockSpec((_ROW_BLOCK, w), lambda i: (i, 0)),
        out_shape=jax.ShapeDtypeStruct((n, w), thetas.dtype),
    )(thetas)


def kernel(thetas, time_derivs, coeff_0, coeff_1, coeff_2, coeff_3):
    cols = np.nonzero(_MASKS[0])[0].astype(np.int32)
    gathered = _masked_gather(thetas, cols)
    sparse_thetas = (gathered,) * _N_OUT
    return sparse_thetas + (coeff_0, coeff_1, coeff_2, coeff_3)
